# baseline (speedup 1.0000x reference)
import functools

import jax
import jax.numpy as jnp
from jax import lax
from jax.experimental import pallas as pl
from jax.experimental.pallas import tpu as pltpu
from jax.experimental.pallas import tpu_sc as plsc

_ALPHA = 0.25
_DELTA = 0.1
_AB = 4096

_NW = 32
_NCG = 512
_CGW = _NCG // _NW


def _focal(ce):
    p = jnp.exp(-ce)
    return _ALPHA * (1.0 - p) * (1.0 - p) * ce


def _cls_body(cls_r, gc_r, go_r, cls_o):
    j = pl.program_id(0)

    @pl.when(j == 0)
    def _():
        cls_o[0, 0] = 0.0

    acc = jnp.zeros((1, _AB), dtype=jnp.float32)
    for bi in range(cls_r.shape[0]):
        go = go_r[bi:bi + 1, :]
        gc = gc_r[bi:bi + 1, :]
        mask_bb = (go == 1).astype(jnp.float32)
        x = cls_r[bi]
        s = jnp.sum(jnp.exp(x), axis=0, keepdims=True)
        oh = jax.lax.broadcasted_iota(jnp.int32, x.shape, 0) == gc
        sel = jnp.sum(jnp.where(oh, x, 0.0), axis=0, keepdims=True)
        ce = jnp.log(s) - sel
        acc += _focal(ce) * mask_bb
    cls_o[0, 0] += jnp.sum(acc)


@functools.partial(jax.jit, static_argnames=("interpret",))
def _cls_sum(clsT, gc2, go2, interpret=False):
    bsz, c, a = clsT.shape
    nj = a // _AB
    return pl.pallas_call(
        _cls_body,
        grid=(nj,),
        in_specs=[
            pl.BlockSpec((bsz, c, _AB), lambda j: (0, 0, j)),
            pl.BlockSpec((bsz, _AB), lambda j: (0, j)),
            pl.BlockSpec((bsz, _AB), lambda j: (0, j)),
        ],
        out_specs=pl.BlockSpec((1, 1), lambda j: (0, 0),
                               memory_space=pltpu.SMEM),
        out_shape=jax.ShapeDtypeStruct((1, 1), jnp.float32),
        compiler_params=pltpu.CompilerParams(
            dimension_semantics=("arbitrary",)),
        interpret=interpret,
    )(clsT, gc2, go2)


def _v(x):
    return jnp.full((16,), x, dtype=jnp.float32)


def _ln(s):
    bits = lax.bitcast_convert_type(s, jnp.int32)
    e = lax.shift_right_logical(bits, 23) - 127
    m = lax.bitcast_convert_type(
        (bits & 0x007FFFFF) | 0x3F800000, jnp.float32)
    t = (m - 1.0) / (m + 1.0)
    t2 = t * t
    poly = 1.0 + t2 * (0.33333334 + t2 * (0.2 + t2 * 0.14285715))
    return e.astype(jnp.float32) * 0.6931472 + 2.0 * t * poly


def _sc_body(tof, tbf, gbf, gof, obj_out, bb_out,
             go_v, to_v, tb_v, gb_v, acc_o, acc_b):
    wid = lax.axis_index("s") * 2 + lax.axis_index("c")
    pltpu.sync_copy(gof.at[pl.ds(wid * (_CGW * 1024), _CGW * 1024)], go_v)

    obj_acc = jnp.zeros((16,), dtype=jnp.float32)
    bb_acc = jnp.zeros((16,), dtype=jnp.float32)
    for b in range(8):
        pltpu.sync_copy(
            tof.at[pl.ds(b * 131072 + wid * (_CGW * 256), _CGW * 256)], to_v)
        pltpu.sync_copy(
            tbf.at[pl.ds(b * 262144 + wid * (_CGW * 512), _CGW * 512)], tb_v)
        pltpu.sync_copy(
            gbf.at[pl.ds(b * 262144 + wid * (_CGW * 512), _CGW * 512)], gb_v)

        def cg_step(cg, carry):
            o_acc, b_acc = carry
            for v in range(8):
                go16 = go_v[pl.ds(cg * 1024 + b * 128 + v * 16, 16)]
                pos = go16 == 1
                mobj = jnp.where(go16 != -1, _v(1.0), _v(0.0))
                mbb = jnp.where(pos, _v(1.0), _v(0.0))

                av = to_v[pl.ds(cg * 256 + v * 16, 16)]
                bv = to_v[pl.ds(cg * 256 + 128 + v * 16, 16)]
                s2 = jnp.exp(av) + jnp.exp(bv)
                sel = jnp.where(pos, bv, av)
                ce = _ln(s2) - sel
                p = jnp.exp(-ce)
                o_acc = o_acc + (_ALPHA * (1.0 - p) * (1.0 - p) * ce) * mobj

                sl = jnp.zeros((16,), dtype=jnp.float32)
                for k in range(4):
                    off = cg * 512 + k * 128 + v * 16
                    d = tb_v[pl.ds(off, 16)] - gb_v[pl.ds(off, 16)]
                    ad = jnp.abs(d)
                    sl = sl + jnp.where(ad < _DELTA, (0.5 / _DELTA) * d * d,
                                        ad - 0.5 * _DELTA)
                b_acc = b_acc + sl * mbb
            return o_acc, b_acc

        obj_acc, bb_acc = lax.fori_loop(0, _CGW, cg_step, (obj_acc, bb_acc))

    acc_o[pl.ds(0, 16)] = obj_acc
    acc_b[pl.ds(0, 16)] = bb_acc
    pltpu.sync_copy(acc_o, obj_out.at[pl.ds(wid * 16, 16)])
    pltpu.sync_copy(acc_b, bb_out.at[pl.ds(wid * 16, 16)])


def _obj_bb_sums(tof, tbf, gbf, gof):
    mesh = plsc.VectorSubcoreMesh(core_axis_name="c", subcore_axis_name="s")
    f = functools.partial(
        pl.kernel,
        out_type=[jax.ShapeDtypeStruct((_NW * 16,), jnp.float32)] * 2,
        mesh=mesh,
        scratch_types=[
            pltpu.VMEM((_CGW * 1024,), jnp.int32),
            pltpu.VMEM((_CGW * 256,), jnp.float32),
            pltpu.VMEM((_CGW * 512,), jnp.float32),
            pltpu.VMEM((_CGW * 512,), jnp.float32),
            pltpu.VMEM((16,), jnp.float32),
            pltpu.VMEM((16,), jnp.float32),
        ],
    )(_sc_body)
    return f(tof, tbf, gbf, gof)


def kernel(targets_bb, targets_cls, targets_obj, gt_targets_bb,
           gt_targets_cls, gt_targets_obj, w_obj, w_cls, w_bb, step,
           interpret=False):
    n = targets_cls.shape[0] * targets_cls.shape[1]
    clsT = jnp.transpose(targets_cls, (0, 2, 1))

    tof = jnp.transpose(jnp.reshape(targets_obj, (8, 512, 128, 2)),
                        (0, 1, 3, 2)).reshape(-1)
    tbf = jnp.transpose(jnp.reshape(targets_bb, (8, 512, 128, 4)),
                        (0, 1, 3, 2)).reshape(-1)
    gbf = jnp.transpose(jnp.reshape(gt_targets_bb, (8, 512, 128, 4)),
                        (0, 1, 3, 2)).reshape(-1)
    gof = jnp.transpose(jnp.reshape(gt_targets_obj, (8, 512, 128)),
                        (1, 0, 2)).reshape(-1)

    cls_s = _cls_sum(clsT, gt_targets_cls, gt_targets_obj,
                     interpret=interpret)
    obj_p, bb_p = _obj_bb_sums(tof, tbf, gbf, gof)

    inv_n = 1.0 / jnp.float32(n)
    cls_loss = cls_s[0, 0] * inv_n * 10000.0
    obj_loss = jnp.sum(obj_p) * inv_n * 5000.0
    bb_loss = jnp.sum(bb_p) * inv_n * 20000.0
    cls_loss = cls_loss * jnp.exp(-w_cls) + w_cls
    obj_loss = obj_loss * jnp.exp(-w_obj) + w_obj
    bb_loss = bb_loss * jnp.exp(-w_bb) + w_bb
    return (cls_loss, obj_loss, bb_loss)


# speedup vs baseline: 1.0590x; 1.0590x over previous
import functools

import jax
import jax.numpy as jnp
from jax import lax
from jax.experimental import pallas as pl
from jax.experimental.pallas import tpu as pltpu
from jax.experimental.pallas import tpu_sc as plsc

_ALPHA = 0.25
_DELTA = 0.1
_AB = 4096

_NW = 32
_NCG = 512
_CGW = _NCG // _NW


def _focal(ce):
    p = jnp.exp(-ce)
    return _ALPHA * (1.0 - p) * (1.0 - p) * ce


def _cls_body(cls_r, gc_r, go_r, cls_o):
    j = pl.program_id(0)

    @pl.when(j == 0)
    def _():
        cls_o[0, 0] = 0.0

    acc = jnp.zeros((1, _AB), dtype=jnp.float32)
    for bi in range(cls_r.shape[0]):
        go = go_r[bi:bi + 1, :]
        gc = gc_r[bi:bi + 1, :]
        mask_bb = (go == 1).astype(jnp.float32)
        x = cls_r[bi]
        s = jnp.sum(jnp.exp(x), axis=0, keepdims=True)
        oh = jax.lax.broadcasted_iota(jnp.int32, x.shape, 0) == gc
        sel = jnp.sum(jnp.where(oh, x, 0.0), axis=0, keepdims=True)
        ce = jnp.log(s) - sel
        acc += _focal(ce) * mask_bb
    cls_o[0, 0] += jnp.sum(acc)


@functools.partial(jax.jit, static_argnames=("interpret",))
def _cls_sum(clsT, gc2, go2, interpret=False):
    bsz, c, a = clsT.shape
    nj = a // _AB
    return pl.pallas_call(
        _cls_body,
        grid=(nj,),
        in_specs=[
            pl.BlockSpec((bsz, c, _AB), lambda j: (0, 0, j)),
            pl.BlockSpec((bsz, _AB), lambda j: (0, j)),
            pl.BlockSpec((bsz, _AB), lambda j: (0, j)),
        ],
        out_specs=pl.BlockSpec((1, 1), lambda j: (0, 0),
                               memory_space=pltpu.SMEM),
        out_shape=jax.ShapeDtypeStruct((1, 1), jnp.float32),
        compiler_params=pltpu.CompilerParams(
            dimension_semantics=("arbitrary",)),
        interpret=interpret,
    )(clsT, gc2, go2)


def _v(x):
    return jnp.full((16,), x, dtype=jnp.float32)


def _ln(s):
    bits = lax.bitcast_convert_type(s, jnp.int32)
    e = lax.shift_right_logical(bits, 23) - 127
    m = lax.bitcast_convert_type(
        (bits & 0x007FFFFF) | 0x3F800000, jnp.float32)
    t = (m - 1.0) / (m + 1.0)
    t2 = t * t
    poly = 1.0 + t2 * (0.33333334 + t2 * (0.2 + t2 * 0.14285715))
    return e.astype(jnp.float32) * 0.6931472 + 2.0 * t * poly


def _sc_body(tof, tbf, gbf, gof, obj_out, bb_out,
             go_v, to_v, tb_v, gb_v, acc_o, acc_b):
    wid = lax.axis_index("s") * 2 + lax.axis_index("c")
    pltpu.sync_copy(gof.at[pl.ds(wid * (_CGW * 1024), _CGW * 1024)], go_v)

    obj_acc = jnp.zeros((16,), dtype=jnp.float32)
    bb_acc = jnp.zeros((16,), dtype=jnp.float32)
    for b in range(8):
        pltpu.sync_copy(
            tof.at[pl.ds(b * 131072 + wid * (_CGW * 256), _CGW * 256)], to_v)
        pltpu.sync_copy(
            tbf.at[pl.ds(b * 262144 + wid * (_CGW * 512), _CGW * 512)], tb_v)
        pltpu.sync_copy(
            gbf.at[pl.ds(b * 262144 + wid * (_CGW * 512), _CGW * 512)], gb_v)

        @plsc.parallel_loop(0, _CGW, 1, unroll=2,
                            carry=(obj_acc, bb_acc))
        def cg_step(cg, carry):
            o_acc, b_acc = carry
            for v in range(8):
                go16 = go_v[pl.ds(cg * 1024 + b * 128 + v * 16, 16)]
                pos = go16 == 1

                av = to_v[pl.ds(cg * 256 + v * 16, 16)]
                bv = to_v[pl.ds(cg * 256 + 128 + v * 16, 16)]
                ea = jnp.exp(av)
                eb = jnp.exp(bv)
                s2 = ea + eb
                sel = jnp.where(pos, bv, av)
                p = jnp.where(pos, eb, ea) / s2
                ce = _ln(s2) - sel
                focal = _ALPHA * (1.0 - p) * (1.0 - p) * ce
                o_acc = o_acc + jnp.where(go16 != -1, focal, _v(0.0))

                sl = jnp.zeros((16,), dtype=jnp.float32)
                for k in range(4):
                    off = cg * 512 + k * 128 + v * 16
                    d = tb_v[pl.ds(off, 16)] - gb_v[pl.ds(off, 16)]
                    ad = jnp.abs(d)
                    sl = sl + jnp.where(ad < _DELTA, (0.5 / _DELTA) * d * d,
                                        ad - 0.5 * _DELTA)
                b_acc = b_acc + jnp.where(pos, sl, _v(0.0))
            return o_acc, b_acc

        obj_acc, bb_acc = cg_step

    acc_o[pl.ds(0, 16)] = obj_acc
    acc_b[pl.ds(0, 16)] = bb_acc
    pltpu.sync_copy(acc_o, obj_out.at[pl.ds(wid * 16, 16)])
    pltpu.sync_copy(acc_b, bb_out.at[pl.ds(wid * 16, 16)])


def _obj_bb_sums(tof, tbf, gbf, gof):
    mesh = plsc.VectorSubcoreMesh(core_axis_name="c", subcore_axis_name="s")
    f = functools.partial(
        pl.kernel,
        out_type=[jax.ShapeDtypeStruct((_NW * 16,), jnp.float32)] * 2,
        mesh=mesh,
        scratch_types=[
            pltpu.VMEM((_CGW * 1024,), jnp.int32),
            pltpu.VMEM((_CGW * 256,), jnp.float32),
            pltpu.VMEM((_CGW * 512,), jnp.float32),
            pltpu.VMEM((_CGW * 512,), jnp.float32),
            pltpu.VMEM((16,), jnp.float32),
            pltpu.VMEM((16,), jnp.float32),
        ],
    )(_sc_body)
    return f(tof, tbf, gbf, gof)


def kernel(targets_bb, targets_cls, targets_obj, gt_targets_bb,
           gt_targets_cls, gt_targets_obj, w_obj, w_cls, w_bb, step,
           interpret=False):
    n = targets_cls.shape[0] * targets_cls.shape[1]
    clsT = jnp.transpose(targets_cls, (0, 2, 1))

    tof = jnp.transpose(jnp.reshape(targets_obj, (8, 512, 128, 2)),
                        (0, 1, 3, 2)).reshape(-1)
    tbf = jnp.transpose(jnp.reshape(targets_bb, (8, 512, 128, 4)),
                        (0, 1, 3, 2)).reshape(-1)
    gbf = jnp.transpose(jnp.reshape(gt_targets_bb, (8, 512, 128, 4)),
                        (0, 1, 3, 2)).reshape(-1)
    gof = jnp.transpose(jnp.reshape(gt_targets_obj, (8, 512, 128)),
                        (1, 0, 2)).reshape(-1)

    cls_s = _cls_sum(clsT, gt_targets_cls, gt_targets_obj,
                     interpret=interpret)
    obj_p, bb_p = _obj_bb_sums(tof, tbf, gbf, gof)

    inv_n = 1.0 / jnp.float32(n)
    cls_loss = cls_s[0, 0] * inv_n * 10000.0
    obj_loss = jnp.sum(obj_p) * inv_n * 5000.0
    bb_loss = jnp.sum(bb_p) * inv_n * 20000.0
    cls_loss = cls_loss * jnp.exp(-w_cls) + w_cls
    obj_loss = obj_loss * jnp.exp(-w_obj) + w_obj
    bb_loss = bb_loss * jnp.exp(-w_bb) + w_bb
    return (cls_loss, obj_loss, bb_loss)


# speedup vs baseline: 1.2563x; 1.1862x over previous
import functools

import jax
import jax.numpy as jnp
from jax import lax
from jax.experimental import pallas as pl
from jax.experimental.pallas import tpu as pltpu
from jax.experimental.pallas import tpu_sc as plsc

_ALPHA = 0.25
_DELTA = 0.1
_AB = 4096

_NW = 32
_NCG = 512
_CGW = _NCG // _NW


def _focal(ce):
    p = jnp.exp(-ce)
    return _ALPHA * (1.0 - p) * (1.0 - p) * ce


def _cls_body(cls_r, gc_r, go_r, cls_o):
    j = pl.program_id(0)

    @pl.when(j == 0)
    def _():
        cls_o[0, 0] = 0.0

    acc = jnp.zeros((1, _AB), dtype=jnp.float32)
    for bi in range(cls_r.shape[0]):
        go = go_r[bi:bi + 1, :]
        gc = gc_r[bi:bi + 1, :]
        mask_bb = (go == 1).astype(jnp.float32)
        x = cls_r[bi]
        s = jnp.sum(jnp.exp(x), axis=0, keepdims=True)
        oh = jax.lax.broadcasted_iota(jnp.int32, x.shape, 0) == gc
        sel = jnp.sum(jnp.where(oh, x, 0.0), axis=0, keepdims=True)
        ce = jnp.log(s) - sel
        acc += _focal(ce) * mask_bb
    cls_o[0, 0] += jnp.sum(acc)


@functools.partial(jax.jit, static_argnames=("interpret",))
def _cls_sum(clsT, gc2, go2, interpret=False):
    bsz, c, a = clsT.shape
    nj = a // _AB
    return pl.pallas_call(
        _cls_body,
        grid=(nj,),
        in_specs=[
            pl.BlockSpec((bsz, c, _AB), lambda j: (0, 0, j)),
            pl.BlockSpec((bsz, _AB), lambda j: (0, j)),
            pl.BlockSpec((bsz, _AB), lambda j: (0, j)),
        ],
        out_specs=pl.BlockSpec((1, 1), lambda j: (0, 0),
                               memory_space=pltpu.SMEM),
        out_shape=jax.ShapeDtypeStruct((1, 1), jnp.float32),
        compiler_params=pltpu.CompilerParams(
            dimension_semantics=("arbitrary",)),
        interpret=interpret,
    )(clsT, gc2, go2)


def _v(x):
    return jnp.full((16,), x, dtype=jnp.float32)


def _ln(s):
    bits = lax.bitcast_convert_type(s, jnp.int32)
    e = lax.shift_right_logical(bits, 23) - 127
    m = lax.bitcast_convert_type(
        (bits & 0x007FFFFF) | 0x3F800000, jnp.float32)
    t = (m - 1.0) / (m + 1.0)
    t2 = t * t
    poly = 1.0 + t2 * (0.33333334 + t2 * (0.2 + t2 * 0.14285715))
    return e.astype(jnp.float32) * 0.6931472 + 2.0 * t * poly


def _sc_body(tof, tbf, gbf, gof, obj_out, bb_out,
             go_v, to_v, tb_v, gb_v, acc_o, acc_b):
    wid = lax.axis_index("s") * 2 + lax.axis_index("c")
    pltpu.sync_copy(gof.at[pl.ds(wid * (_CGW * 1024), _CGW * 1024)], go_v)

    obj_acc = jnp.zeros((16,), dtype=jnp.float32)
    bb_acc = jnp.zeros((16,), dtype=jnp.float32)
    for b in range(8):
        pltpu.sync_copy(
            tof.at[pl.ds(b * 131072 + wid * (_CGW * 256), _CGW * 256)], to_v)
        pltpu.sync_copy(
            tbf.at[pl.ds(b * 262144 + wid * (_CGW * 512), _CGW * 512)], tb_v)
        pltpu.sync_copy(
            gbf.at[pl.ds(b * 262144 + wid * (_CGW * 512), _CGW * 512)], gb_v)

        @plsc.parallel_loop(0, _CGW * 8, 1, unroll=4,
                            carry=(obj_acc, bb_acc))
        def cg_step(i, carry):
            o_acc, b_acc = carry
            cg = i // 8
            v = i % 8
            go16 = go_v[pl.ds(cg * 1024 + b * 128 + v * 16, 16)]
            pos = go16 == 1

            av = to_v[pl.ds(cg * 256 + v * 16, 16)]
            bv = to_v[pl.ds(cg * 256 + 128 + v * 16, 16)]
            ea = jnp.exp(av)
            eb = jnp.exp(bv)
            s2 = ea + eb
            sel = jnp.where(pos, bv, av)
            p = jnp.where(pos, eb, ea) / s2
            ce = _ln(s2) - sel
            focal = _ALPHA * (1.0 - p) * (1.0 - p) * ce
            o_acc = o_acc + jnp.where(go16 != -1, focal, _v(0.0))

            sl = jnp.zeros((16,), dtype=jnp.float32)
            for k in range(4):
                off = cg * 512 + k * 128 + v * 16
                d = tb_v[pl.ds(off, 16)] - gb_v[pl.ds(off, 16)]
                ad = jnp.abs(d)
                sl = sl + jnp.where(ad < _DELTA, (0.5 / _DELTA) * d * d,
                                    ad - 0.5 * _DELTA)
            b_acc = b_acc + jnp.where(pos, sl, _v(0.0))
            return o_acc, b_acc

        obj_acc, bb_acc = cg_step

    acc_o[pl.ds(0, 16)] = obj_acc
    acc_b[pl.ds(0, 16)] = bb_acc
    pltpu.sync_copy(acc_o, obj_out.at[pl.ds(wid * 16, 16)])
    pltpu.sync_copy(acc_b, bb_out.at[pl.ds(wid * 16, 16)])


def _obj_bb_sums(tof, tbf, gbf, gof):
    mesh = plsc.VectorSubcoreMesh(core_axis_name="c", subcore_axis_name="s")
    f = functools.partial(
        pl.kernel,
        out_type=[jax.ShapeDtypeStruct((_NW * 16,), jnp.float32)] * 2,
        mesh=mesh,
        scratch_types=[
            pltpu.VMEM((_CGW * 1024,), jnp.int32),
            pltpu.VMEM((_CGW * 256,), jnp.float32),
            pltpu.VMEM((_CGW * 512,), jnp.float32),
            pltpu.VMEM((_CGW * 512,), jnp.float32),
            pltpu.VMEM((16,), jnp.float32),
            pltpu.VMEM((16,), jnp.float32),
        ],
    )(_sc_body)
    return f(tof, tbf, gbf, gof)


def kernel(targets_bb, targets_cls, targets_obj, gt_targets_bb,
           gt_targets_cls, gt_targets_obj, w_obj, w_cls, w_bb, step,
           interpret=False):
    n = targets_cls.shape[0] * targets_cls.shape[1]
    clsT = jnp.transpose(targets_cls, (0, 2, 1))

    tof = jnp.transpose(jnp.reshape(targets_obj, (8, 512, 128, 2)),
                        (0, 1, 3, 2)).reshape(-1)
    tbf = jnp.transpose(jnp.reshape(targets_bb, (8, 512, 128, 4)),
                        (0, 1, 3, 2)).reshape(-1)
    gbf = jnp.transpose(jnp.reshape(gt_targets_bb, (8, 512, 128, 4)),
                        (0, 1, 3, 2)).reshape(-1)
    gof = jnp.transpose(jnp.reshape(gt_targets_obj, (8, 512, 128)),
                        (1, 0, 2)).reshape(-1)

    cls_s = _cls_sum(clsT, gt_targets_cls, gt_targets_obj,
                     interpret=interpret)
    obj_p, bb_p = _obj_bb_sums(tof, tbf, gbf, gof)

    inv_n = 1.0 / jnp.float32(n)
    cls_loss = cls_s[0, 0] * inv_n * 10000.0
    obj_loss = jnp.sum(obj_p) * inv_n * 5000.0
    bb_loss = jnp.sum(bb_p) * inv_n * 20000.0
    cls_loss = cls_loss * jnp.exp(-w_cls) + w_cls
    obj_loss = obj_loss * jnp.exp(-w_obj) + w_obj
    bb_loss = bb_loss * jnp.exp(-w_bb) + w_bb
    return (cls_loss, obj_loss, bb_loss)
